# Initial kernel scaffold; baseline (speedup 1.0000x reference)
import jax
import jax.numpy as jnp
from jax.experimental import pallas as pl


def kernel(x, edge_index, edge_type, rel_embed, w_loop, w_in, w_out, w_rel, loop_rel, bias, gamma, beta):
    raise NotImplementedError("write your pallas kernel here")


# speedup vs baseline: 11.0014x; 11.0014x over previous
import functools

import jax
import jax.numpy as jnp
from jax import lax
from jax.experimental import pallas as pl
from jax.experimental.pallas import tpu as pltpu
from jax.experimental.pallas import tpu_sc as plsc

LANES = 16
TILES = 16
K = 128


def _deg_kernel_body(NP, RPT, CH, D, src_hbm, out_hbm, deg_sh, idx_v, ones_v):
    c = lax.axis_index("c")
    s = lax.axis_index("s")

    def fill(val):
        def body(i, carry):
            for j in range(D // LANES):
                ones_v[i, pl.ds(j * LANES, LANES)] = jnp.full(
                    (LANES,), val, jnp.float32)
            return carry
        lax.fori_loop(0, K, body, 0)

    fill(0.0)
    for b in range(RPT // K):
        pltpu.sync_copy(ones_v, deg_sh.at[pl.ds(s * RPT + b * K, K)])
    fill(1.0)
    plsc.subcore_barrier()

    pltpu.sync_copy(src_hbm.at[c, s], idx_v)

    def chunk(ch, carry):
        pltpu.sync_copy(ones_v, deg_sh.at[idx_v.at[ch]], add=True)
        return carry

    lax.fori_loop(0, CH, chunk, 0)
    plsc.subcore_barrier()
    pltpu.sync_copy(deg_sh.at[pl.ds(s * RPT, RPT)],
                    out_hbm.at[c, pl.ds(s * RPT, RPT)])


def _agg_kernel_body(NP, RPT, CH, D,
                     idx_hbm, xs_hbm, rel_hbm, out_hbm,
                     agg_sh, idx_v, xs_v, rel_v, sem1, sem2):
    c = lax.axis_index("c")
    s = lax.axis_index("s")

    def zero_row(i, carry):
        for j in range(D // LANES):
            xs_v[i, pl.ds(j * LANES, LANES)] = jnp.zeros((LANES,), jnp.float32)
        return carry

    lax.fori_loop(0, K, zero_row, 0)
    for b in range(RPT // K):
        pltpu.sync_copy(xs_v, agg_sh.at[pl.ds(s * RPT + b * K, K)])
    plsc.subcore_barrier()

    def chunk(ch, carry):
        pltpu.sync_copy(idx_hbm.at[c, s, ch], idx_v)
        cp1 = pltpu.async_copy(xs_hbm.at[idx_v.at[0]], xs_v, sem1)
        cp2 = pltpu.async_copy(rel_hbm.at[idx_v.at[1]], rel_v, sem2)
        cp1.wait()
        cp2.wait()

        def mul_row(i, inner):
            for j in range(D // LANES):
                sl = pl.ds(j * LANES, LANES)
                xs_v[i, sl] = xs_v[i, sl] * rel_v[i, sl]
            return inner

        lax.fori_loop(0, K, mul_row, 0)
        pltpu.sync_copy(xs_v, agg_sh.at[idx_v.at[2]], add=True)
        return carry

    lax.fori_loop(0, CH, chunk, 0)
    plsc.subcore_barrier()
    pltpu.sync_copy(agg_sh.at[pl.ds(s * RPT, RPT)],
                    out_hbm.at[c, pl.ds(s * RPT, RPT)])


def _scale_body(deg_ref, x_ref, xs_ref, dinv_ref):
    d = deg_ref[0]
    dinv = jnp.where(d > 0.5, lax.rsqrt(d), 0.0)
    dinvb = jnp.broadcast_to(dinv[:, :, None], x_ref.shape)
    dinv_ref[0] = dinvb
    xs_ref[0] = x_ref[...] * dinvb


def _final_body(N, agg_ref, dinv_ref, x_ref, rel_ref, w_in_ref, w_out_ref,
                w_loop_ref, w_rel_ref, loop_rel_ref, bias_ref, gamma_ref,
                beta_ref, out_ref, rel_out_ref):
    a_in = agg_ref[0, :N, :] * dinv_ref[0, :N, :]
    a_out = agg_ref[1, :N, :] * dinv_ref[1, :N, :]
    xl = x_ref[...] * loop_rel_ref[...]
    h = (jnp.dot(a_in, w_in_ref[...], preferred_element_type=jnp.float32)
         + jnp.dot(a_out, w_out_ref[...], preferred_element_type=jnp.float32)
         + jnp.dot(xl, w_loop_ref[...], preferred_element_type=jnp.float32))
    h = h * (1.0 / 3.0) + bias_ref[...]
    mu = jnp.mean(h, axis=0, keepdims=True)
    var = jnp.mean((h - mu) * (h - mu), axis=0, keepdims=True)
    out_ref[...] = (h - mu) * lax.rsqrt(var + 1e-5) * gamma_ref[...] + beta_ref[...]
    rel_out_ref[...] = jnp.dot(rel_ref[...], w_rel_ref[...],
                               preferred_element_type=jnp.float32)


def kernel(x, edge_index, edge_type, rel_embed, w_loop, w_in, w_out, w_rel,
           loop_rel, bias, gamma, beta):
    N, D = x.shape
    ne = edge_index.shape[1] // 2
    NR = rel_embed.shape[0]

    NP = ((N + TILES * K - 1) // (TILES * K)) * (TILES * K)
    RPT = NP // TILES
    CH = (ne + TILES * K - 1) // (TILES * K)
    EP = CH * K * TILES

    mesh = plsc.VectorSubcoreMesh(core_axis_name="c", subcore_axis_name="s",
                                  num_cores=2, num_subcores=TILES)

    src = edge_index[0]
    dst = edge_index[1]
    pad = EP - ne

    def pad_to(a, val):
        return jnp.concatenate(
            [a, jnp.full((pad,), val, jnp.int32)]).reshape(TILES, CH, K)

    SRC = jnp.stack([pad_to(src[:ne], N), pad_to(src[ne:], N)])
    DST = jnp.stack([pad_to(dst[:ne], N), pad_to(dst[ne:], N)])
    ETY = jnp.stack([pad_to(edge_type[:ne], 0), pad_to(edge_type[ne:], 0)])
    SRCo = SRC + (jnp.arange(2, dtype=jnp.int32) * NP)[:, None, None, None]
    IDX = jnp.stack([SRCo, ETY, DST], axis=3)

    deg_call = functools.partial(
        pl.kernel,
        out_type=jax.ShapeDtypeStruct((2, NP, D), jnp.float32),
        mesh=mesh,
        scratch_types=[
            pltpu.VMEM_SHARED((NP, D), jnp.float32),
            pltpu.VMEM((CH, K), jnp.int32),
            pltpu.VMEM((K, D), jnp.float32),
        ],
    )(functools.partial(_deg_kernel_body, NP, RPT, CH, D))
    deg_full = deg_call(SRC)
    deg = deg_full[:, :, 0].reshape(2, NP // K, K)

    G = NP // K
    x_pad = jnp.pad(x, ((0, NP - N), (0, 0))).reshape(G, K, D)
    xs4, dinv4 = pl.pallas_call(
        _scale_body,
        grid=(2,),
        in_specs=[
            pl.BlockSpec((1, G, K), lambda d: (d, 0, 0)),
            pl.BlockSpec((G, K, D), lambda d: (0, 0, 0)),
        ],
        out_specs=[
            pl.BlockSpec((1, G, K, D), lambda d: (d, 0, 0, 0)),
            pl.BlockSpec((1, G, K, D), lambda d: (d, 0, 0, 0)),
        ],
        out_shape=[
            jax.ShapeDtypeStruct((2, G, K, D), jnp.float32),
            jax.ShapeDtypeStruct((2, G, K, D), jnp.float32),
        ],
    )(deg, x_pad)
    xs_flat = xs4.reshape(2 * NP, D)
    dinv_full = dinv4.reshape(2, NP, D)

    agg_call = functools.partial(
        pl.kernel,
        out_type=jax.ShapeDtypeStruct((2, NP, D), jnp.float32),
        mesh=mesh,
        scratch_types=[
            pltpu.VMEM_SHARED((NP, D), jnp.float32),
            pltpu.VMEM((3, K), jnp.int32),
            pltpu.VMEM((K, D), jnp.float32),
            pltpu.VMEM((K, D), jnp.float32),
            pltpu.SemaphoreType.DMA,
            pltpu.SemaphoreType.DMA,
        ],
    )(functools.partial(_agg_kernel_body, NP, RPT, CH, D))
    agg = agg_call(IDX, xs_flat, rel_embed)

    out, rel_out = pl.pallas_call(
        functools.partial(_final_body, N),
        out_shape=[
            jax.ShapeDtypeStruct((N, D), jnp.float32),
            jax.ShapeDtypeStruct((NR, D), jnp.float32),
        ],
    )(agg, dinv_full, x, rel_embed, w_in, w_out, w_loop, w_rel,
      loop_rel.reshape(1, D), bias.reshape(1, D), gamma.reshape(1, D),
      beta.reshape(1, D))
    return out, rel_out


# speedup vs baseline: 14.9335x; 1.3574x over previous
import functools

import jax
import jax.numpy as jnp
from jax import lax
from jax.experimental import pallas as pl
from jax.experimental.pallas import tpu as pltpu
from jax.experimental.pallas import tpu_sc as plsc

LANES = 16
TILES = 16
K = 128


def _deg_kernel_body(NP, RPT, CH, D, src_hbm, out_hbm, deg_sh, idx_v, ones_v):
    c = lax.axis_index("c")
    s = lax.axis_index("s")

    def fill(val):
        def body(i, carry):
            for j in range(D // LANES):
                ones_v[i, pl.ds(j * LANES, LANES)] = jnp.full(
                    (LANES,), val, jnp.float32)
            return carry
        lax.fori_loop(0, K, body, 0)

    fill(0.0)
    for b in range(RPT // K):
        pltpu.sync_copy(ones_v, deg_sh.at[pl.ds(s * RPT + b * K, K)])
    fill(1.0)
    plsc.subcore_barrier()

    pltpu.sync_copy(src_hbm.at[c, s], idx_v)

    def chunk(ch, carry):
        pltpu.sync_copy(ones_v, deg_sh.at[idx_v.at[ch]], add=True)
        return carry

    lax.fori_loop(0, CH, chunk, 0)
    plsc.subcore_barrier()
    pltpu.sync_copy(deg_sh.at[pl.ds(s * RPT, RPT)],
                    out_hbm.at[c, pl.ds(s * RPT, RPT)])


def _agg_kernel_body(NP, RPT, CH, KC, D,
                     idx_hbm, xs_hbm, rel_hbm, out_hbm,
                     agg_sh, idx_v, didx_v, xs_v, rel_v,
                     semx0, semx1, semr0, semr1, sems0, sems1):
    c = lax.axis_index("c")
    s = lax.axis_index("s")
    semx = (semx0, semx1)
    semr = (semr0, semr1)
    sems = (sems0, sems1)
    CH2 = CH // 2

    def zero_row(i, carry):
        for j in range(D // LANES):
            xs_v[0, i, pl.ds(j * LANES, LANES)] = jnp.zeros(
                (LANES,), jnp.float32)
        return carry

    lax.fori_loop(0, KC, zero_row, 0)
    for b in range(RPT // KC):
        pltpu.sync_copy(xs_v.at[0], agg_sh.at[pl.ds(s * RPT + b * KC, KC)])
    plsc.subcore_barrier()

    def gx(b, g):
        return pltpu.async_copy(xs_hbm.at[idx_v.at[b, 0]], xs_v.at[b],
                                semx[b])

    def gr(b, g):
        return pltpu.async_copy(rel_hbm.at[idx_v.at[b, 1]], rel_v.at[b],
                                semr[b])

    for b in range(2):
        pltpu.sync_copy(idx_hbm.at[c, s, b], idx_v.at[b])
        gx(b, b)
        gr(b, b)

    def pipeline(i, carry):
        more = i < CH2 - 1
        for b in range(2):
            g = 2 * i + b
            pltpu.make_async_copy(xs_hbm.at[idx_v.at[b, 0]], xs_v.at[b],
                                  semx[b]).wait()
            pltpu.make_async_copy(rel_hbm.at[idx_v.at[b, 1]], rel_v.at[b],
                                  semr[b]).wait()
            for q in range(KC // LANES):
                sl = pl.ds(q * LANES, LANES)
                didx_v[b, sl] = idx_v[b, 2, sl]

            def mul_row(r, inner):
                for j in range(D // LANES):
                    sl2 = pl.ds(j * LANES, LANES)
                    rel_v[b, r, sl2] = xs_v[b, r, sl2] * rel_v[b, r, sl2]
                return inner

            lax.fori_loop(0, KC, mul_row, 0)
            scat = pltpu.async_copy(rel_v.at[b], agg_sh.at[didx_v.at[b]],
                                    sems[b], add=True)

            @pl.when(more)
            def _prefetch():
                pltpu.sync_copy(idx_hbm.at[c, s, g + 2], idx_v.at[b])
                gx(b, g + 2)

            scat.wait()

            @pl.when(more)
            def _prefetch_rel():
                gr(b, g + 2)

        return carry

    lax.fori_loop(0, CH2, pipeline, 0)
    plsc.subcore_barrier()
    pltpu.sync_copy(agg_sh.at[pl.ds(s * RPT, RPT)],
                    out_hbm.at[c, pl.ds(s * RPT, RPT)])


def _scale_body(deg_ref, x_ref, xs_ref, dinv_ref):
    d = deg_ref[0]
    dinv = jnp.where(d > 0.5, lax.rsqrt(d), 0.0)
    dinvb = jnp.broadcast_to(dinv[:, :, None], x_ref.shape)
    dinv_ref[0] = dinvb
    xs_ref[0] = x_ref[...] * dinvb


def _final_body(N, agg_ref, dinv_ref, x_ref, rel_ref, w_in_ref, w_out_ref,
                w_loop_ref, w_rel_ref, loop_rel_ref, bias_ref, gamma_ref,
                beta_ref, out_ref, rel_out_ref):
    a_in = agg_ref[0, :N, :] * dinv_ref[0, :N, :]
    a_out = agg_ref[1, :N, :] * dinv_ref[1, :N, :]
    xl = x_ref[...] * loop_rel_ref[...]
    h = (jnp.dot(a_in, w_in_ref[...], preferred_element_type=jnp.float32)
         + jnp.dot(a_out, w_out_ref[...], preferred_element_type=jnp.float32)
         + jnp.dot(xl, w_loop_ref[...], preferred_element_type=jnp.float32))
    h = h * (1.0 / 3.0) + bias_ref[...]
    mu = jnp.mean(h, axis=0, keepdims=True)
    var = jnp.mean((h - mu) * (h - mu), axis=0, keepdims=True)
    out_ref[...] = (h - mu) * lax.rsqrt(var + 1e-5) * gamma_ref[...] + beta_ref[...]
    rel_out_ref[...] = jnp.dot(rel_ref[...], w_rel_ref[...],
                               preferred_element_type=jnp.float32)


def kernel(x, edge_index, edge_type, rel_embed, w_loop, w_in, w_out, w_rel,
           loop_rel, bias, gamma, beta):
    N, D = x.shape
    ne = edge_index.shape[1] // 2
    NR = rel_embed.shape[0]

    NP = ((N + TILES * K - 1) // (TILES * K)) * (TILES * K)
    RPT = NP // TILES
    CH = (ne + TILES * K - 1) // (TILES * K)
    EP = CH * K * TILES
    KC = 80
    CHC = 2 * ((ne + 2 * TILES * KC - 1) // (2 * TILES * KC))
    EPC = CHC * KC * TILES

    mesh = plsc.VectorSubcoreMesh(core_axis_name="c", subcore_axis_name="s",
                                  num_cores=2, num_subcores=TILES)

    src = edge_index[0]
    dst = edge_index[1]

    def pad_to(a, val, ep, ch, k):
        return jnp.concatenate(
            [a, jnp.full((ep - ne,), val, jnp.int32)]).reshape(TILES, ch, k)

    SRC = jnp.stack([pad_to(src[:ne], N, EP, CH, K),
                     pad_to(src[ne:], N, EP, CH, K)])
    SRCc = jnp.stack([pad_to(src[:ne], N, EPC, CHC, KC),
                      pad_to(src[ne:], N, EPC, CHC, KC)])
    DSTc = jnp.stack([pad_to(dst[:ne], N, EPC, CHC, KC),
                      pad_to(dst[ne:], N, EPC, CHC, KC)])
    ETYc = jnp.stack([pad_to(edge_type[:ne], 0, EPC, CHC, KC),
                      pad_to(edge_type[ne:], 0, EPC, CHC, KC)])
    SRCo = SRCc + (jnp.arange(2, dtype=jnp.int32) * NP)[:, None, None, None]
    IDX = jnp.stack([SRCo, ETYc, DSTc], axis=3)

    deg_call = functools.partial(
        pl.kernel,
        out_type=jax.ShapeDtypeStruct((2, NP, D), jnp.float32),
        mesh=mesh,
        scratch_types=[
            pltpu.VMEM_SHARED((NP, D), jnp.float32),
            pltpu.VMEM((CH, K), jnp.int32),
            pltpu.VMEM((K, D), jnp.float32),
        ],
    )(functools.partial(_deg_kernel_body, NP, RPT, CH, D))
    deg_full = deg_call(SRC)
    deg = deg_full[:, :, 0].reshape(2, NP // K, K)

    G = NP // K
    x_pad = jnp.pad(x, ((0, NP - N), (0, 0))).reshape(G, K, D)
    xs4, dinv4 = pl.pallas_call(
        _scale_body,
        grid=(2,),
        in_specs=[
            pl.BlockSpec((1, G, K), lambda d: (d, 0, 0)),
            pl.BlockSpec((G, K, D), lambda d: (0, 0, 0)),
        ],
        out_specs=[
            pl.BlockSpec((1, G, K, D), lambda d: (d, 0, 0, 0)),
            pl.BlockSpec((1, G, K, D), lambda d: (d, 0, 0, 0)),
        ],
        out_shape=[
            jax.ShapeDtypeStruct((2, G, K, D), jnp.float32),
            jax.ShapeDtypeStruct((2, G, K, D), jnp.float32),
        ],
    )(deg, x_pad)
    xs_flat = xs4.reshape(2 * NP, D)
    dinv_full = dinv4.reshape(2, NP, D)

    agg_call = functools.partial(
        pl.kernel,
        out_type=jax.ShapeDtypeStruct((2, NP, D), jnp.float32),
        mesh=mesh,
        scratch_types=[
            pltpu.VMEM_SHARED((NP, D), jnp.float32),
            pltpu.VMEM((2, 3, KC), jnp.int32),
            pltpu.VMEM((2, KC), jnp.int32),
            pltpu.VMEM((2, KC, D), jnp.float32),
            pltpu.VMEM((2, KC, D), jnp.float32),
            pltpu.SemaphoreType.DMA,
            pltpu.SemaphoreType.DMA,
            pltpu.SemaphoreType.DMA,
            pltpu.SemaphoreType.DMA,
            pltpu.SemaphoreType.DMA,
            pltpu.SemaphoreType.DMA,
        ],
    )(functools.partial(_agg_kernel_body, NP, RPT, CHC, KC, D))
    agg = agg_call(IDX, xs_flat, rel_embed)

    out, rel_out = pl.pallas_call(
        functools.partial(_final_body, N),
        out_shape=[
            jax.ShapeDtypeStruct((N, D), jnp.float32),
            jax.ShapeDtypeStruct((NR, D), jnp.float32),
        ],
    )(agg, dinv_full, x, rel_embed, w_in, w_out, w_loop, w_rel,
      loop_rel.reshape(1, D), bias.reshape(1, D), gamma.reshape(1, D),
      beta.reshape(1, D))
    return out, rel_out
